# baseline (speedup 1.0000x reference)
import functools
import math

import jax
import jax.numpy as jnp
from jax import lax
from jax.experimental import pallas as pl
from jax.experimental.pallas import tpu as pltpu
from jax.experimental.pallas import tpu_sc as plsc

N_ATOMS = 32768
NB = 1024
H = 256
HH = 128
H4 = 64
RBF = 16
EPS = 1e-5

NC = 2
NS = 16
NW = NC * NS
PW = N_ATOMS // NW
RPW = PW // 128
BPW = NB // NS

NCH = 2
NA2 = N_ATOMS // NCH
PW2 = NA2 // NW
RPW2 = PW2 // 128

TILE = 2048
NT = N_ATOMS // TILE
NT2 = NA2 // TILE

_MESH = functools.partial(
    plsc.VectorSubcoreMesh, core_axis_name="c", subcore_axis_name="s")


def _ln(x, g, b):
    mu = jnp.mean(x, axis=-1, keepdims=True)
    var = jnp.mean((x - mu) ** 2, axis=-1, keepdims=True)
    return (x - mu) * lax.rsqrt(var + EPS) * g + b


def _wid():
    return lax.axis_index("s") * NC + lax.axis_index("c")


def _sc_stats_body(posp_h, bid_h, zeros_h, out_h, idx_v, rows_v, zb_v, shared):
    c = lax.axis_index("c")
    s = lax.axis_index("s")
    wid = s * NC + c
    pltpu.sync_copy(zeros_h.at[pl.ds(s * BPW, BPW)], zb_v)
    pltpu.sync_copy(zb_v, shared.at[pl.ds(s * BPW, BPW)])
    plsc.subcore_barrier()
    pltpu.sync_copy(bid_h.at[pl.ds(wid * RPW, RPW)], idx_v)
    pltpu.sync_copy(posp_h.at[pl.ds(wid * PW, PW)], rows_v)
    for j in range(RPW):
        pltpu.sync_copy(rows_v.at[pl.ds(j * 128, 128)],
                        shared.at[idx_v.at[j]], add=True)
    plsc.subcore_barrier()
    pltpu.sync_copy(shared.at[pl.ds(s * BPW, BPW)], zb_v)
    pltpu.sync_copy(zb_v, out_h.at[pl.ds(c * NB + s * BPW, BPW)])


def _sc_stats(posp16, bid2, zeros16):
    return pl.kernel(
        _sc_stats_body,
        out_type=jax.ShapeDtypeStruct((NC * NB, 16), jnp.float32),
        mesh=_MESH(),
        compiler_params=pltpu.CompilerParams(use_tc_tiling_on_sc=False),
        scratch_types=[
            pltpu.VMEM((RPW, 128), jnp.int32),
            pltpu.VMEM((PW, 16), jnp.float32),
            pltpu.VMEM((BPW, 16), jnp.float32),
            pltpu.VMEM_SHARED((NB, 16), jnp.float32),
        ],
    )(posp16, bid2, zeros16)


def _centq_body(stats_ref, bf_ref, wq_ref, bq_ref,
                cent_ref, qlo_ref, qhi_ref):
    stats = stats_ref[:NB, :] + stats_ref[NB:, :]
    cnt = jnp.maximum(stats[:, 0:1], 1.0)
    lane = lax.broadcasted_iota(jnp.int32, (NB, 16), 1)
    keep = (lane >= 1) & (lane <= 3)
    cent_ref[...] = jnp.where(keep, stats / cnt, 0.0)
    q = jnp.dot(bf_ref[...], wq_ref[...],
                preferred_element_type=jnp.float32) + bq_ref[...]
    qlo_ref[...] = q[:, :HH]
    qhi_ref[...] = q[:, HH:]


def _make_gather2_body(off):
    orow = off // 128

    def body(cent_h, qlo_h, qhi_h, bid_h, centg_h, qlog_h, qhig_h,
             idx_v, crow_v, l0_v, l1_v, h0_v, h1_v, csem, gsem, osem):
        wid = _wid()
        base = wid * PW2
        pltpu.sync_copy(bid_h.at[pl.ds(orow + wid * RPW2, RPW2)], idx_v)
        for j in range(RPW2):
            pltpu.async_copy(cent_h.at[idx_v.at[j]],
                             crow_v.at[pl.ds(j * 128, 128)], csem)
        lb = (l0_v, l1_v)
        hb = (h0_v, h1_v)
        pltpu.async_copy(qlo_h.at[idx_v.at[0]], l0_v, gsem)
        pltpu.async_copy(qhi_h.at[idx_v.at[0]], h0_v, gsem)
        for j in range(RPW2):
            if j >= 1:
                pltpu.make_async_copy(
                    lb[(j - 1) % 2],
                    qlog_h.at[pl.ds(base + (j - 1) * 128, 128)], osem).wait()
                pltpu.make_async_copy(
                    hb[(j - 1) % 2],
                    qhig_h.at[pl.ds(base + (j - 1) * 128, 128)], osem).wait()
            if j + 1 < RPW2:
                pltpu.async_copy(qlo_h.at[idx_v.at[j + 1]],
                                 lb[(j + 1) % 2], gsem)
                pltpu.async_copy(qhi_h.at[idx_v.at[j + 1]],
                                 hb[(j + 1) % 2], gsem)
            pltpu.make_async_copy(qlo_h.at[idx_v.at[j]], lb[j % 2], gsem).wait()
            pltpu.make_async_copy(qhi_h.at[idx_v.at[j]], hb[j % 2], gsem).wait()
            pltpu.async_copy(lb[j % 2],
                             qlog_h.at[pl.ds(base + j * 128, 128)], osem)
            pltpu.async_copy(hb[j % 2],
                             qhig_h.at[pl.ds(base + j * 128, 128)], osem)
        for j in range(RPW2):
            pltpu.make_async_copy(cent_h.at[idx_v.at[j]],
                                  crow_v.at[pl.ds(j * 128, 128)], csem).wait()
        pltpu.sync_copy(crow_v, centg_h.at[pl.ds(base, PW2)])
        pltpu.make_async_copy(
            lb[(RPW2 - 1) % 2],
            qlog_h.at[pl.ds(base + (RPW2 - 1) * 128, 128)], osem).wait()
        pltpu.make_async_copy(
            hb[(RPW2 - 1) % 2],
            qhig_h.at[pl.ds(base + (RPW2 - 1) * 128, 128)], osem).wait()

    return body


def _sc_gather2(cent16, qlo, qhi, bid2, off):
    return pl.kernel(
        _make_gather2_body(off),
        out_type=[jax.ShapeDtypeStruct((NA2, 16), jnp.float32),
                  jax.ShapeDtypeStruct((NA2, HH), jnp.float32),
                  jax.ShapeDtypeStruct((NA2, HH), jnp.float32)],
        mesh=_MESH(),
        compiler_params=pltpu.CompilerParams(use_tc_tiling_on_sc=False),
        scratch_types=[
            pltpu.VMEM((RPW2, 128), jnp.int32),
            pltpu.VMEM((PW2, 16), jnp.float32),
            pltpu.VMEM((128, HH), jnp.float32),
            pltpu.VMEM((128, HH), jnp.float32),
            pltpu.VMEM((128, HH), jnp.float32),
            pltpu.VMEM((128, HH), jnp.float32),
            pltpu.SemaphoreType.DMA,
            pltpu.SemaphoreType.DMA,
            pltpu.SemaphoreType.DMA,
        ],
    )(cent16, qlo, qhi, bid2)


def _atoms_body(af_ref, pos_ref, centg_ref, qlog_ref, qhig_ref,
                cen_ref, wid_ref, wg_ref, bg_ref,
                wka_ref, wkb_ref, bk_ref, wva_ref, wvb_ref, bv_ref,
                evlo_ref, evhi_ref, epk_ref):
    pos = pos_ref[...]
    cg = centg_ref[...]
    dx = pos[:, 0:1] - cg[:, 1:2]
    dy = pos[:, 1:2] - cg[:, 2:3]
    dz = pos[:, 2:3] - cg[:, 3:4]
    dist = jnp.sqrt(dx * dx + dy * dy + dz * dz)
    d = dist - cen_ref[...]
    rbf = jnp.exp(-(d * d) / (2.0 * wid_ref[...] * wid_ref[...]))
    geom = jnp.dot(rbf, wg_ref[...],
                   preferred_element_type=jnp.float32) + bg_ref[...]

    af = af_ref[...]
    k = (jnp.dot(af, wka_ref[...], preferred_element_type=jnp.float32)
         + jnp.dot(geom, wkb_ref[...], preferred_element_type=jnp.float32)
         + bk_ref[...])
    v = (jnp.dot(af, wva_ref[...], preferred_element_type=jnp.float32)
         + jnp.dot(geom, wvb_ref[...], preferred_element_type=jnp.float32)
         + bv_ref[...])

    s = (jnp.sum(qlog_ref[...] * k[:, :HH], axis=1)
         + jnp.sum(qhig_ref[...] * k[:, HH:], axis=1)) * (1.0 / math.sqrt(H))
    e = jnp.exp(s)
    ecol = e[:, None]
    evlo_ref[...] = ecol * v[:, :HH]
    evhi_ref[...] = ecol * v[:, HH:]
    e2 = e.reshape(TILE // 8, 8)
    grp = lax.broadcasted_iota(jnp.int32, (8, 128), 1) // 16
    row = lax.broadcasted_iota(jnp.int32, (8, 128), 0)
    sel = jnp.where(grp == row, 1.0, 0.0)
    epk_ref[...] = jnp.dot(e2, sel, preferred_element_type=jnp.float32)


def _make_ctx_body(off):
    orow = off // 128

    def body(evlo_h, evhi_h, e16_h, bid_h, zlohi_h, ze_h,
             acclo_h, acchi_h, acce_h,
             idx_v, l0_v, l1_v, h0_v, h1_v, erow_v, zb_v, zbe_v,
             shlo, shhi, she, lsem, ssem):
        c = lax.axis_index("c")
        s = lax.axis_index("s")
        wid = s * NC + c
        base = wid * PW2
        pltpu.sync_copy(zlohi_h.at[pl.ds(s * BPW, BPW)], zb_v)
        pltpu.sync_copy(zb_v, shlo.at[pl.ds(s * BPW, BPW)])
        pltpu.sync_copy(zb_v, shhi.at[pl.ds(s * BPW, BPW)])
        pltpu.sync_copy(ze_h.at[pl.ds(s * BPW, BPW)], zbe_v)
        pltpu.sync_copy(zbe_v, she.at[pl.ds(s * BPW, BPW)])
        pltpu.sync_copy(bid_h.at[pl.ds(orow + wid * RPW2, RPW2)], idx_v)
        pltpu.sync_copy(e16_h.at[pl.ds(base, PW2)], erow_v)
        plsc.subcore_barrier()
        lb = (l0_v, l1_v)
        hb = (h0_v, h1_v)
        pltpu.async_copy(evlo_h.at[pl.ds(base, 128)], l0_v, lsem)
        pltpu.async_copy(evhi_h.at[pl.ds(base, 128)], h0_v, lsem)
        for j in range(RPW2):
            pltpu.sync_copy(erow_v.at[pl.ds(j * 128, 128)],
                            she.at[idx_v.at[j]], add=True)
            if j >= 1:
                pltpu.make_async_copy(lb[(j - 1) % 2],
                                      shlo.at[idx_v.at[j - 1]], ssem).wait()
                pltpu.make_async_copy(hb[(j - 1) % 2],
                                      shhi.at[idx_v.at[j - 1]], ssem).wait()
            if j + 1 < RPW2:
                pltpu.async_copy(evlo_h.at[pl.ds(base + (j + 1) * 128, 128)],
                                 lb[(j + 1) % 2], lsem)
                pltpu.async_copy(evhi_h.at[pl.ds(base + (j + 1) * 128, 128)],
                                 hb[(j + 1) % 2], lsem)
            pltpu.make_async_copy(evlo_h.at[pl.ds(base + j * 128, 128)],
                                  lb[j % 2], lsem).wait()
            pltpu.make_async_copy(evhi_h.at[pl.ds(base + j * 128, 128)],
                                  hb[j % 2], lsem).wait()
            pltpu.async_copy(lb[j % 2], shlo.at[idx_v.at[j]], ssem, add=True)
            pltpu.async_copy(hb[j % 2], shhi.at[idx_v.at[j]], ssem, add=True)
        pltpu.make_async_copy(lb[(RPW2 - 1) % 2],
                              shlo.at[idx_v.at[RPW2 - 1]], ssem).wait()
        pltpu.make_async_copy(hb[(RPW2 - 1) % 2],
                              shhi.at[idx_v.at[RPW2 - 1]], ssem).wait()
        plsc.subcore_barrier()
        pltpu.sync_copy(shlo.at[pl.ds(s * BPW, BPW)], zb_v)
        pltpu.sync_copy(zb_v, acclo_h.at[pl.ds(c * NB + s * BPW, BPW)])
        pltpu.sync_copy(shhi.at[pl.ds(s * BPW, BPW)], zb_v)
        pltpu.sync_copy(zb_v, acchi_h.at[pl.ds(c * NB + s * BPW, BPW)])
        pltpu.sync_copy(she.at[pl.ds(s * BPW, BPW)], zbe_v)
        pltpu.sync_copy(zbe_v, acce_h.at[pl.ds(c * NB + s * BPW, BPW)])

    return body


def _sc_ctx(evlo, evhi, e16, bid2, zlohi, ze16, off):
    return pl.kernel(
        _make_ctx_body(off),
        out_type=[jax.ShapeDtypeStruct((NC * NB, HH), jnp.float32),
                  jax.ShapeDtypeStruct((NC * NB, HH), jnp.float32),
                  jax.ShapeDtypeStruct((NC * NB, 16), jnp.float32)],
        mesh=_MESH(),
        compiler_params=pltpu.CompilerParams(use_tc_tiling_on_sc=False),
        scratch_types=[
            pltpu.VMEM((RPW2, 128), jnp.int32),
            pltpu.VMEM((128, HH), jnp.float32),
            pltpu.VMEM((128, HH), jnp.float32),
            pltpu.VMEM((128, HH), jnp.float32),
            pltpu.VMEM((128, HH), jnp.float32),
            pltpu.VMEM((PW2, 16), jnp.float32),
            pltpu.VMEM((BPW, HH), jnp.float32),
            pltpu.VMEM((BPW, 16), jnp.float32),
            pltpu.VMEM_SHARED((NB, HH), jnp.float32),
            pltpu.VMEM_SHARED((NB, HH), jnp.float32),
            pltpu.VMEM_SHARED((NB, 16), jnp.float32),
            pltpu.SemaphoreType.DMA,
            pltpu.SemaphoreType.DMA,
        ],
    )(evlo, evhi, e16, bid2, zlohi, ze16)


def _upd_body(alo0_ref, ahi0_ref, ae0_ref, alo1_ref, ahi1_ref, ae1_ref,
              wc1_ref, bc1_ref, wc2_ref, bc2_ref, updlo_ref, updhi_ref):
    ae = ae0_ref[:NB, 0:1] + ae0_ref[NB:, 0:1] \
        + ae1_ref[:NB, 0:1] + ae1_ref[NB:, 0:1]
    den = jnp.maximum(ae, 1e-30)
    ctx = jnp.concatenate(
        [alo0_ref[:NB, :] + alo0_ref[NB:, :]
         + alo1_ref[:NB, :] + alo1_ref[NB:, :],
         ahi0_ref[:NB, :] + ahi0_ref[NB:, :]
         + ahi1_ref[:NB, :] + ahi1_ref[NB:, :]], axis=1) / den
    h1 = jax.nn.relu(jnp.dot(ctx, wc1_ref[...],
                             preferred_element_type=jnp.float32) + bc1_ref[...])
    upd = jnp.dot(h1, wc2_ref[...],
                  preferred_element_type=jnp.float32) + bc2_ref[...]
    updlo_ref[...] = upd[:, :HH]
    updhi_ref[...] = upd[:, HH:]


def _make_gather1_body(off):
    orow = off // 128

    def body(updlo_h, updhi_h, bid_h, uglo_h, ughi_h,
             idx_v, l0_v, l1_v, h0_v, h1_v, gsem, osem):
        wid = _wid()
        base = wid * PW2
        pltpu.sync_copy(bid_h.at[pl.ds(orow + wid * RPW2, RPW2)], idx_v)
        lb = (l0_v, l1_v)
        hb = (h0_v, h1_v)
        pltpu.async_copy(updlo_h.at[idx_v.at[0]], l0_v, gsem)
        pltpu.async_copy(updhi_h.at[idx_v.at[0]], h0_v, gsem)
        for j in range(RPW2):
            if j >= 1:
                pltpu.make_async_copy(
                    lb[(j - 1) % 2],
                    uglo_h.at[pl.ds(base + (j - 1) * 128, 128)], osem).wait()
                pltpu.make_async_copy(
                    hb[(j - 1) % 2],
                    ughi_h.at[pl.ds(base + (j - 1) * 128, 128)], osem).wait()
            if j + 1 < RPW2:
                pltpu.async_copy(updlo_h.at[idx_v.at[j + 1]],
                                 lb[(j + 1) % 2], gsem)
                pltpu.async_copy(updhi_h.at[idx_v.at[j + 1]],
                                 hb[(j + 1) % 2], gsem)
            pltpu.make_async_copy(updlo_h.at[idx_v.at[j]], lb[j % 2], gsem).wait()
            pltpu.make_async_copy(updhi_h.at[idx_v.at[j]], hb[j % 2], gsem).wait()
            pltpu.async_copy(lb[j % 2],
                             uglo_h.at[pl.ds(base + j * 128, 128)], osem)
            pltpu.async_copy(hb[j % 2],
                             ughi_h.at[pl.ds(base + j * 128, 128)], osem)
        pltpu.make_async_copy(
            lb[(RPW2 - 1) % 2],
            uglo_h.at[pl.ds(base + (RPW2 - 1) * 128, 128)], osem).wait()
        pltpu.make_async_copy(
            hb[(RPW2 - 1) % 2],
            ughi_h.at[pl.ds(base + (RPW2 - 1) * 128, 128)], osem).wait()

    return body


def _sc_gather1(updlo, updhi, bid2, off):
    return pl.kernel(
        _make_gather1_body(off),
        out_type=[jax.ShapeDtypeStruct((NA2, HH), jnp.float32),
                  jax.ShapeDtypeStruct((NA2, HH), jnp.float32)],
        mesh=_MESH(),
        compiler_params=pltpu.CompilerParams(use_tc_tiling_on_sc=False),
        scratch_types=[
            pltpu.VMEM((RPW2, 128), jnp.int32),
            pltpu.VMEM((128, HH), jnp.float32),
            pltpu.VMEM((128, HH), jnp.float32),
            pltpu.VMEM((128, HH), jnp.float32),
            pltpu.VMEM((128, HH), jnp.float32),
            pltpu.SemaphoreType.DMA,
            pltpu.SemaphoreType.DMA,
        ],
    )(updlo, updhi, bid2)


def _final_body(af_ref, uglo_ref, ughi_ref,
                wf1_ref, bf1_ref, wf2_ref, bf2_ref,
                g1_ref, b1_ref, g2_ref, b2_ref, out_ref):
    upd = jnp.concatenate([uglo_ref[...], ughi_ref[...]], axis=1)
    x = _ln(af_ref[...] + upd, g1_ref[...], b1_ref[...])
    f = jax.nn.relu(jnp.dot(x, wf1_ref[...],
                            preferred_element_type=jnp.float32) + bf1_ref[...])
    f = jnp.dot(f, wf2_ref[...],
                preferred_element_type=jnp.float32) + bf2_ref[...]
    out_ref[...] = _ln(x + f, g2_ref[...], b2_ref[...])


def kernel(atom_features, atom_positions, block_features, params, block_id):
    p = params
    af = atom_features
    posp16 = jnp.concatenate(
        [jnp.ones((N_ATOMS, 1), jnp.float32), atom_positions,
         jnp.zeros((N_ATOMS, 12), jnp.float32)], axis=1)
    bid2 = block_id.reshape(N_ATOMS // 128, 128)
    zeros16 = jnp.zeros((NB, 16), jnp.float32)
    zlohi = jnp.zeros((NB, HH), jnp.float32)

    cen_pad = jnp.zeros((128,), jnp.float32).at[:RBF].set(p['centers'])
    wid_pad = jnp.ones((128,), jnp.float32).at[:RBF].set(p['widths'])
    wg_pad = jnp.zeros((128, 128), jnp.float32).at[:RBF, :H4].set(p['Wg'])
    bg_pad = jnp.zeros((128,), jnp.float32).at[:H4].set(p['bg'])
    wka, wkb = p['Wk'][:H], jnp.zeros((128, H), jnp.float32).at[:H4].set(p['Wk'][H:])
    wva, wvb = p['Wv'][:H], jnp.zeros((128, H), jnp.float32).at[:H4].set(p['Wv'][H:])

    stats = _sc_stats(posp16, bid2, zeros16)

    cent16, qlo, qhi = pl.pallas_call(
        _centq_body,
        in_specs=[pl.BlockSpec((NC * NB, 16), lambda: (0, 0)),
                  pl.BlockSpec((NB, H), lambda: (0, 0)),
                  pl.BlockSpec((H, H), lambda: (0, 0)),
                  pl.BlockSpec((H,), lambda: (0,))],
        out_specs=[pl.BlockSpec((NB, 16), lambda: (0, 0)),
                   pl.BlockSpec((NB, HH), lambda: (0, 0)),
                   pl.BlockSpec((NB, HH), lambda: (0, 0))],
        out_shape=[jax.ShapeDtypeStruct((NB, 16), jnp.float32),
                   jax.ShapeDtypeStruct((NB, HH), jnp.float32),
                   jax.ShapeDtypeStruct((NB, HH), jnp.float32)],
    )(stats, block_features, p['Wq'], p['bq'])

    vec = lambda n: pl.BlockSpec((n,), lambda i: (0,))
    mat = lambda a, b: pl.BlockSpec((a, b), lambda i: (0, 0))

    accs = []
    for ci in range(NCH):
        off = ci * NA2
        centg, qlog, qhig = _sc_gather2(cent16, qlo, qhi, bid2, off)

        ctile = lambda c: pl.BlockSpec((TILE, c), lambda i: (i, 0))
        ftile = lambda c: pl.BlockSpec(
            (TILE, c), lambda i, _o=off // TILE: (i + _o, 0))

        evlo, evhi, epk = pl.pallas_call(
            _atoms_body,
            grid=(NT2,),
            in_specs=[ftile(H), ftile(3), ctile(16), ctile(HH), ctile(HH),
                      vec(128), vec(128), mat(128, 128), vec(128),
                      mat(H, H), mat(128, H), vec(H),
                      mat(H, H), mat(128, H), vec(H)],
            out_specs=[ctile(HH), ctile(HH),
                       pl.BlockSpec((TILE // 8, 128), lambda i: (i, 0))],
            out_shape=[jax.ShapeDtypeStruct((NA2, HH), jnp.float32),
                       jax.ShapeDtypeStruct((NA2, HH), jnp.float32),
                       jax.ShapeDtypeStruct((NA2 // 8, 128), jnp.float32)],
        )(af, atom_positions, centg, qlog, qhig,
          cen_pad, wid_pad, wg_pad, bg_pad,
          wka, wkb, p['bk'], wva, wvb, p['bv'])

        e16 = epk.reshape(NA2, 16)
        accs.append(_sc_ctx(evlo, evhi, e16, bid2, zlohi, zeros16, off))

    (alo0, ahi0, ae0), (alo1, ahi1, ae1) = accs

    updlo, updhi = pl.pallas_call(
        _upd_body,
        in_specs=[pl.BlockSpec((NC * NB, HH), lambda: (0, 0)),
                  pl.BlockSpec((NC * NB, HH), lambda: (0, 0)),
                  pl.BlockSpec((NC * NB, 16), lambda: (0, 0)),
                  pl.BlockSpec((NC * NB, HH), lambda: (0, 0)),
                  pl.BlockSpec((NC * NB, HH), lambda: (0, 0)),
                  pl.BlockSpec((NC * NB, 16), lambda: (0, 0)),
                  pl.BlockSpec((H, H), lambda: (0, 0)),
                  pl.BlockSpec((H,), lambda: (0,)),
                  pl.BlockSpec((H, H), lambda: (0, 0)),
                  pl.BlockSpec((H,), lambda: (0,))],
        out_specs=[pl.BlockSpec((NB, HH), lambda: (0, 0)),
                   pl.BlockSpec((NB, HH), lambda: (0, 0))],
        out_shape=[jax.ShapeDtypeStruct((NB, HH), jnp.float32),
                   jax.ShapeDtypeStruct((NB, HH), jnp.float32)],
    )(alo0, ahi0, ae0, alo1, ahi1, ae1,
      p['Wc1'], p['bc1'], p['Wc2'], p['bc2'])

    outs = []
    for ci in range(NCH):
        off = ci * NA2
        uglo, ughi = _sc_gather1(updlo, updhi, bid2, off)

        ctile = lambda c: pl.BlockSpec((TILE, c), lambda i: (i, 0))
        ftile = lambda c: pl.BlockSpec(
            (TILE, c), lambda i, _o=off // TILE: (i + _o, 0))
        outs.append(pl.pallas_call(
            _final_body,
            grid=(NT2,),
            in_specs=[ftile(H), ctile(HH), ctile(HH),
                      mat(H, 2 * H), vec(2 * H), mat(2 * H, H), vec(H),
                      vec(H), vec(H), vec(H), vec(H)],
            out_specs=ctile(H),
            out_shape=jax.ShapeDtypeStruct((NA2, H), jnp.float32),
        )(af, uglo, ughi,
          p['Wf1'], p['bf1'], p['Wf2'], p['bf2'],
          p['g1'], p['b1'], p['g2'], p['b2']))
    return jnp.concatenate(outs, axis=0)


# speedup vs baseline: 1.0631x; 1.0631x over previous
import functools
import math

import jax
import jax.numpy as jnp
from jax import lax
from jax.experimental import pallas as pl
from jax.experimental.pallas import tpu as pltpu
from jax.experimental.pallas import tpu_sc as plsc

N_ATOMS = 32768
NB = 1024
H = 256
HH = 128
H4 = 64
RBF = 16
EPS = 1e-5

NC = 2
NS = 16
NW = NC * NS
PW = N_ATOMS // NW
RPW = PW // 128
BPW = NB // NS

NCH = 2
NA2 = N_ATOMS // NCH
PW2 = NA2 // NW
RPW2 = PW2 // 128

TILE = 2048
NT = N_ATOMS // TILE
NT2 = NA2 // TILE

_MESH = functools.partial(
    plsc.VectorSubcoreMesh, core_axis_name="c", subcore_axis_name="s")


def _ln(x, g, b):
    mu = jnp.mean(x, axis=-1, keepdims=True)
    var = jnp.mean((x - mu) ** 2, axis=-1, keepdims=True)
    return (x - mu) * lax.rsqrt(var + EPS) * g + b


def _wid():
    return lax.axis_index("s") * NC + lax.axis_index("c")


def _sc_stats_body(posp_h, bid_h, zeros_h, out_h, idx_v, rows_v, zb_v, shared):
    c = lax.axis_index("c")
    s = lax.axis_index("s")
    wid = s * NC + c
    pltpu.sync_copy(zeros_h.at[pl.ds(s * BPW, BPW)], zb_v)
    pltpu.sync_copy(zb_v, shared.at[pl.ds(s * BPW, BPW)])
    plsc.subcore_barrier()
    pltpu.sync_copy(bid_h.at[pl.ds(wid * RPW, RPW)], idx_v)
    pltpu.sync_copy(posp_h.at[pl.ds(wid * PW, PW)], rows_v)
    for j in range(RPW):
        pltpu.sync_copy(rows_v.at[pl.ds(j * 128, 128)],
                        shared.at[idx_v.at[j]], add=True)
    plsc.subcore_barrier()
    pltpu.sync_copy(shared.at[pl.ds(s * BPW, BPW)], zb_v)
    pltpu.sync_copy(zb_v, out_h.at[pl.ds(c * NB + s * BPW, BPW)])


def _sc_stats(posp16, bid2, zeros16):
    return pl.kernel(
        _sc_stats_body,
        out_type=jax.ShapeDtypeStruct((NC * NB, 16), jnp.float32),
        mesh=_MESH(),
        compiler_params=pltpu.CompilerParams(use_tc_tiling_on_sc=False),
        scratch_types=[
            pltpu.VMEM((RPW, 128), jnp.int32),
            pltpu.VMEM((PW, 16), jnp.float32),
            pltpu.VMEM((BPW, 16), jnp.float32),
            pltpu.VMEM_SHARED((NB, 16), jnp.float32),
        ],
    )(posp16, bid2, zeros16)


def _centq_body(stats_ref, bf_ref, wq_ref, bq_ref,
                cent_ref, qlo_ref, qhi_ref):
    stats = stats_ref[:NB, :] + stats_ref[NB:, :]
    cnt = jnp.maximum(stats[:, 0:1], 1.0)
    lane = lax.broadcasted_iota(jnp.int32, (NB, 16), 1)
    keep = (lane >= 1) & (lane <= 3)
    cent_ref[...] = jnp.where(keep, stats / cnt, 0.0)
    q = jnp.dot(bf_ref[...], wq_ref[...],
                preferred_element_type=jnp.float32) + bq_ref[...]
    qlo_ref[...] = q[:, :HH]
    qhi_ref[...] = q[:, HH:]


def _make_gather2_body(off):
    orow = off // 128

    def body(cent_h, qlo_h, qhi_h, bid_h, centg_h, qlog_h, qhig_h,
             idx_v, crow_v, l0_v, l1_v, h0_v, h1_v, csem, gsem, osem):
        wid = _wid()
        base = wid * PW2
        pltpu.sync_copy(bid_h.at[pl.ds(orow + wid * RPW2, RPW2)], idx_v)
        for j in range(RPW2):
            pltpu.async_copy(cent_h.at[idx_v.at[j]],
                             crow_v.at[pl.ds(j * 128, 128)], csem)
        lb = (l0_v, l1_v)
        hb = (h0_v, h1_v)
        pltpu.async_copy(qlo_h.at[idx_v.at[0]], l0_v, gsem)
        pltpu.async_copy(qhi_h.at[idx_v.at[0]], h0_v, gsem)
        for j in range(RPW2):
            if j >= 1:
                pltpu.make_async_copy(
                    lb[(j - 1) % 2],
                    qlog_h.at[pl.ds(base + (j - 1) * 128, 128)], osem).wait()
                pltpu.make_async_copy(
                    hb[(j - 1) % 2],
                    qhig_h.at[pl.ds(base + (j - 1) * 128, 128)], osem).wait()
            if j + 1 < RPW2:
                pltpu.async_copy(qlo_h.at[idx_v.at[j + 1]],
                                 lb[(j + 1) % 2], gsem)
                pltpu.async_copy(qhi_h.at[idx_v.at[j + 1]],
                                 hb[(j + 1) % 2], gsem)
            pltpu.make_async_copy(qlo_h.at[idx_v.at[j]], lb[j % 2], gsem).wait()
            pltpu.make_async_copy(qhi_h.at[idx_v.at[j]], hb[j % 2], gsem).wait()
            pltpu.async_copy(lb[j % 2],
                             qlog_h.at[pl.ds(base + j * 128, 128)], osem)
            pltpu.async_copy(hb[j % 2],
                             qhig_h.at[pl.ds(base + j * 128, 128)], osem)
        for j in range(RPW2):
            pltpu.make_async_copy(cent_h.at[idx_v.at[j]],
                                  crow_v.at[pl.ds(j * 128, 128)], csem).wait()
        pltpu.sync_copy(crow_v, centg_h.at[pl.ds(base, PW2)])
        pltpu.make_async_copy(
            lb[(RPW2 - 1) % 2],
            qlog_h.at[pl.ds(base + (RPW2 - 1) * 128, 128)], osem).wait()
        pltpu.make_async_copy(
            hb[(RPW2 - 1) % 2],
            qhig_h.at[pl.ds(base + (RPW2 - 1) * 128, 128)], osem).wait()

    return body


def _sc_gather2(cent16, qlo, qhi, bid2, off):
    return pl.kernel(
        _make_gather2_body(off),
        out_type=[jax.ShapeDtypeStruct((NA2, 16), jnp.float32),
                  jax.ShapeDtypeStruct((NA2, HH), jnp.float32),
                  jax.ShapeDtypeStruct((NA2, HH), jnp.float32)],
        mesh=_MESH(),
        compiler_params=pltpu.CompilerParams(use_tc_tiling_on_sc=False),
        scratch_types=[
            pltpu.VMEM((RPW2, 128), jnp.int32),
            pltpu.VMEM((PW2, 16), jnp.float32),
            pltpu.VMEM((128, HH), jnp.float32),
            pltpu.VMEM((128, HH), jnp.float32),
            pltpu.VMEM((128, HH), jnp.float32),
            pltpu.VMEM((128, HH), jnp.float32),
            pltpu.SemaphoreType.DMA,
            pltpu.SemaphoreType.DMA,
            pltpu.SemaphoreType.DMA,
        ],
    )(cent16, qlo, qhi, bid2)


def _atoms_body(af_ref, pos_ref, centg_ref, qlog_ref, qhig_ref,
                cen_ref, wid_ref, wg_ref, bg_ref,
                wka_ref, wkb_ref, bk_ref, wva_ref, wvb_ref, bv_ref,
                evlo_ref, evhi_ref, epk_ref):
    pos = pos_ref[...]
    cg = centg_ref[...]
    dx = pos[:, 0:1] - cg[:, 1:2]
    dy = pos[:, 1:2] - cg[:, 2:3]
    dz = pos[:, 2:3] - cg[:, 3:4]
    dist = jnp.sqrt(dx * dx + dy * dy + dz * dz)
    d = dist - cen_ref[...]
    rbf = jnp.exp(-(d * d) / (2.0 * wid_ref[...] * wid_ref[...]))
    geom = jnp.dot(rbf, wg_ref[...],
                   preferred_element_type=jnp.float32) + bg_ref[...]

    af = af_ref[...]
    k = (jnp.dot(af, wka_ref[...], preferred_element_type=jnp.float32)
         + jnp.dot(geom, wkb_ref[...], preferred_element_type=jnp.float32)
         + bk_ref[...])
    v = (jnp.dot(af, wva_ref[...], preferred_element_type=jnp.float32)
         + jnp.dot(geom, wvb_ref[...], preferred_element_type=jnp.float32)
         + bv_ref[...])

    s = (jnp.sum(qlog_ref[...] * k[:, :HH], axis=1)
         + jnp.sum(qhig_ref[...] * k[:, HH:], axis=1)) * (1.0 / math.sqrt(H))
    e = jnp.exp(s)
    ecol = e[:, None]
    evlo_ref[...] = ecol * v[:, :HH]
    evhi_ref[...] = ecol * v[:, HH:]
    e2 = e.reshape(TILE // 8, 8)
    grp = lax.broadcasted_iota(jnp.int32, (8, 128), 1) // 16
    row = lax.broadcasted_iota(jnp.int32, (8, 128), 0)
    sel = jnp.where(grp == row, 1.0, 0.0)
    epk_ref[...] = jnp.dot(e2, sel, preferred_element_type=jnp.float32)


def _make_ctx_body(off):
    orow = off // 128

    def body(evlo_h, evhi_h, e16_h, bid_h, zlohi_h, ze_h,
             acclo_h, acchi_h, acce_h,
             idx_v, l0_v, l1_v, h0_v, h1_v, erow_v, zb_v, zbe_v,
             shlo, shhi, she, lsem, ssem):
        c = lax.axis_index("c")
        s = lax.axis_index("s")
        wid = s * NC + c
        base = wid * PW2
        pltpu.sync_copy(zlohi_h.at[pl.ds(s * BPW, BPW)], zb_v)
        pltpu.sync_copy(zb_v, shlo.at[pl.ds(s * BPW, BPW)])
        pltpu.sync_copy(zb_v, shhi.at[pl.ds(s * BPW, BPW)])
        pltpu.sync_copy(ze_h.at[pl.ds(s * BPW, BPW)], zbe_v)
        pltpu.sync_copy(zbe_v, she.at[pl.ds(s * BPW, BPW)])
        pltpu.sync_copy(bid_h.at[pl.ds(orow + wid * RPW2, RPW2)], idx_v)
        pltpu.sync_copy(e16_h.at[pl.ds(base, PW2)], erow_v)
        plsc.subcore_barrier()
        lb = (l0_v, l1_v)
        hb = (h0_v, h1_v)
        pltpu.async_copy(evlo_h.at[pl.ds(base, 128)], l0_v, lsem)
        pltpu.async_copy(evhi_h.at[pl.ds(base, 128)], h0_v, lsem)
        for j in range(RPW2):
            pltpu.sync_copy(erow_v.at[pl.ds(j * 128, 128)],
                            she.at[idx_v.at[j]], add=True)
            if j >= 1:
                pltpu.make_async_copy(lb[(j - 1) % 2],
                                      shlo.at[idx_v.at[j - 1]], ssem).wait()
                pltpu.make_async_copy(hb[(j - 1) % 2],
                                      shhi.at[idx_v.at[j - 1]], ssem).wait()
            if j + 1 < RPW2:
                pltpu.async_copy(evlo_h.at[pl.ds(base + (j + 1) * 128, 128)],
                                 lb[(j + 1) % 2], lsem)
                pltpu.async_copy(evhi_h.at[pl.ds(base + (j + 1) * 128, 128)],
                                 hb[(j + 1) % 2], lsem)
            pltpu.make_async_copy(evlo_h.at[pl.ds(base + j * 128, 128)],
                                  lb[j % 2], lsem).wait()
            pltpu.make_async_copy(evhi_h.at[pl.ds(base + j * 128, 128)],
                                  hb[j % 2], lsem).wait()
            pltpu.async_copy(lb[j % 2], shlo.at[idx_v.at[j]], ssem, add=True)
            pltpu.async_copy(hb[j % 2], shhi.at[idx_v.at[j]], ssem, add=True)
        pltpu.make_async_copy(lb[(RPW2 - 1) % 2],
                              shlo.at[idx_v.at[RPW2 - 1]], ssem).wait()
        pltpu.make_async_copy(hb[(RPW2 - 1) % 2],
                              shhi.at[idx_v.at[RPW2 - 1]], ssem).wait()
        plsc.subcore_barrier()
        pltpu.sync_copy(shlo.at[pl.ds(s * BPW, BPW)], zb_v)
        pltpu.sync_copy(zb_v, acclo_h.at[pl.ds(c * NB + s * BPW, BPW)])
        pltpu.sync_copy(shhi.at[pl.ds(s * BPW, BPW)], zb_v)
        pltpu.sync_copy(zb_v, acchi_h.at[pl.ds(c * NB + s * BPW, BPW)])
        pltpu.sync_copy(she.at[pl.ds(s * BPW, BPW)], zbe_v)
        pltpu.sync_copy(zbe_v, acce_h.at[pl.ds(c * NB + s * BPW, BPW)])

    return body


def _sc_ctx(evlo, evhi, e16, bid2, zlohi, ze16, off):
    return pl.kernel(
        _make_ctx_body(off),
        out_type=[jax.ShapeDtypeStruct((NC * NB, HH), jnp.float32),
                  jax.ShapeDtypeStruct((NC * NB, HH), jnp.float32),
                  jax.ShapeDtypeStruct((NC * NB, 16), jnp.float32)],
        mesh=_MESH(),
        compiler_params=pltpu.CompilerParams(use_tc_tiling_on_sc=False),
        scratch_types=[
            pltpu.VMEM((RPW2, 128), jnp.int32),
            pltpu.VMEM((128, HH), jnp.float32),
            pltpu.VMEM((128, HH), jnp.float32),
            pltpu.VMEM((128, HH), jnp.float32),
            pltpu.VMEM((128, HH), jnp.float32),
            pltpu.VMEM((PW2, 16), jnp.float32),
            pltpu.VMEM((BPW, HH), jnp.float32),
            pltpu.VMEM((BPW, 16), jnp.float32),
            pltpu.VMEM_SHARED((NB, HH), jnp.float32),
            pltpu.VMEM_SHARED((NB, HH), jnp.float32),
            pltpu.VMEM_SHARED((NB, 16), jnp.float32),
            pltpu.SemaphoreType.DMA,
            pltpu.SemaphoreType.DMA,
        ],
    )(evlo, evhi, e16, bid2, zlohi, ze16)


def _upd_body(alo0_ref, ahi0_ref, ae0_ref, alo1_ref, ahi1_ref, ae1_ref,
              wc1_ref, bc1_ref, wc2_ref, bc2_ref, updlo_ref, updhi_ref):
    ae = ae0_ref[:NB, 0:1] + ae0_ref[NB:, 0:1] \
        + ae1_ref[:NB, 0:1] + ae1_ref[NB:, 0:1]
    den = jnp.maximum(ae, 1e-30)
    ctx = jnp.concatenate(
        [alo0_ref[:NB, :] + alo0_ref[NB:, :]
         + alo1_ref[:NB, :] + alo1_ref[NB:, :],
         ahi0_ref[:NB, :] + ahi0_ref[NB:, :]
         + ahi1_ref[:NB, :] + ahi1_ref[NB:, :]], axis=1) / den
    h1 = jax.nn.relu(jnp.dot(ctx, wc1_ref[...],
                             preferred_element_type=jnp.float32) + bc1_ref[...])
    upd = jnp.dot(h1, wc2_ref[...],
                  preferred_element_type=jnp.float32) + bc2_ref[...]
    updlo_ref[...] = upd[:, :HH]
    updhi_ref[...] = upd[:, HH:]


def _make_gather1_body(off, pw, rpw):
    orow = off // 128

    def body(updlo_h, updhi_h, bid_h, uglo_h, ughi_h,
             idx_v, l0_v, l1_v, h0_v, h1_v, gsem, osem):
        wid = _wid()
        base = wid * pw
        pltpu.sync_copy(bid_h.at[pl.ds(orow + wid * rpw, rpw)], idx_v)
        lb = (l0_v, l1_v)
        hb = (h0_v, h1_v)
        pltpu.async_copy(updlo_h.at[idx_v.at[0]], l0_v, gsem)
        pltpu.async_copy(updhi_h.at[idx_v.at[0]], h0_v, gsem)
        for j in range(rpw):
            if j >= 1:
                pltpu.make_async_copy(
                    lb[(j - 1) % 2],
                    uglo_h.at[pl.ds(base + (j - 1) * 128, 128)], osem).wait()
                pltpu.make_async_copy(
                    hb[(j - 1) % 2],
                    ughi_h.at[pl.ds(base + (j - 1) * 128, 128)], osem).wait()
            if j + 1 < rpw:
                pltpu.async_copy(updlo_h.at[idx_v.at[j + 1]],
                                 lb[(j + 1) % 2], gsem)
                pltpu.async_copy(updhi_h.at[idx_v.at[j + 1]],
                                 hb[(j + 1) % 2], gsem)
            pltpu.make_async_copy(updlo_h.at[idx_v.at[j]], lb[j % 2], gsem).wait()
            pltpu.make_async_copy(updhi_h.at[idx_v.at[j]], hb[j % 2], gsem).wait()
            pltpu.async_copy(lb[j % 2],
                             uglo_h.at[pl.ds(base + j * 128, 128)], osem)
            pltpu.async_copy(hb[j % 2],
                             ughi_h.at[pl.ds(base + j * 128, 128)], osem)
        pltpu.make_async_copy(
            lb[(rpw - 1) % 2],
            uglo_h.at[pl.ds(base + (rpw - 1) * 128, 128)], osem).wait()
        pltpu.make_async_copy(
            hb[(rpw - 1) % 2],
            ughi_h.at[pl.ds(base + (rpw - 1) * 128, 128)], osem).wait()

    return body


def _sc_gather1(updlo, updhi, bid2, off, na, pw, rpw):
    return pl.kernel(
        _make_gather1_body(off, pw, rpw),
        out_type=[jax.ShapeDtypeStruct((na, HH), jnp.float32),
                  jax.ShapeDtypeStruct((na, HH), jnp.float32)],
        mesh=_MESH(),
        compiler_params=pltpu.CompilerParams(use_tc_tiling_on_sc=False),
        scratch_types=[
            pltpu.VMEM((rpw, 128), jnp.int32),
            pltpu.VMEM((128, HH), jnp.float32),
            pltpu.VMEM((128, HH), jnp.float32),
            pltpu.VMEM((128, HH), jnp.float32),
            pltpu.VMEM((128, HH), jnp.float32),
            pltpu.SemaphoreType.DMA,
            pltpu.SemaphoreType.DMA,
        ],
    )(updlo, updhi, bid2)


def _final_body(af_ref, uglo_ref, ughi_ref,
                wf1_ref, bf1_ref, wf2_ref, bf2_ref,
                g1_ref, b1_ref, g2_ref, b2_ref, out_ref):
    upd = jnp.concatenate([uglo_ref[...], ughi_ref[...]], axis=1)
    x = _ln(af_ref[...] + upd, g1_ref[...], b1_ref[...])
    f = jax.nn.relu(jnp.dot(x, wf1_ref[...],
                            preferred_element_type=jnp.float32) + bf1_ref[...])
    f = jnp.dot(f, wf2_ref[...],
                preferred_element_type=jnp.float32) + bf2_ref[...]
    out_ref[...] = _ln(x + f, g2_ref[...], b2_ref[...])


def kernel(atom_features, atom_positions, block_features, params, block_id):
    p = params
    af = atom_features
    posp16 = jnp.concatenate(
        [jnp.ones((N_ATOMS, 1), jnp.float32), atom_positions,
         jnp.zeros((N_ATOMS, 12), jnp.float32)], axis=1)
    bid2 = block_id.reshape(N_ATOMS // 128, 128)
    zeros16 = jnp.zeros((NB, 16), jnp.float32)
    zlohi = jnp.zeros((NB, HH), jnp.float32)

    cen_pad = jnp.zeros((128,), jnp.float32).at[:RBF].set(p['centers'])
    wid_pad = jnp.ones((128,), jnp.float32).at[:RBF].set(p['widths'])
    wg_pad = jnp.zeros((128, 128), jnp.float32).at[:RBF, :H4].set(p['Wg'])
    bg_pad = jnp.zeros((128,), jnp.float32).at[:H4].set(p['bg'])
    wka, wkb = p['Wk'][:H], jnp.zeros((128, H), jnp.float32).at[:H4].set(p['Wk'][H:])
    wva, wvb = p['Wv'][:H], jnp.zeros((128, H), jnp.float32).at[:H4].set(p['Wv'][H:])

    stats = _sc_stats(posp16, bid2, zeros16)

    cent16, qlo, qhi = pl.pallas_call(
        _centq_body,
        in_specs=[pl.BlockSpec((NC * NB, 16), lambda: (0, 0)),
                  pl.BlockSpec((NB, H), lambda: (0, 0)),
                  pl.BlockSpec((H, H), lambda: (0, 0)),
                  pl.BlockSpec((H,), lambda: (0,))],
        out_specs=[pl.BlockSpec((NB, 16), lambda: (0, 0)),
                   pl.BlockSpec((NB, HH), lambda: (0, 0)),
                   pl.BlockSpec((NB, HH), lambda: (0, 0))],
        out_shape=[jax.ShapeDtypeStruct((NB, 16), jnp.float32),
                   jax.ShapeDtypeStruct((NB, HH), jnp.float32),
                   jax.ShapeDtypeStruct((NB, HH), jnp.float32)],
    )(stats, block_features, p['Wq'], p['bq'])

    vec = lambda n: pl.BlockSpec((n,), lambda i: (0,))
    mat = lambda a, b: pl.BlockSpec((a, b), lambda i: (0, 0))

    accs = []
    for ci in range(NCH):
        off = ci * NA2
        centg, qlog, qhig = _sc_gather2(cent16, qlo, qhi, bid2, off)

        ctile = lambda c: pl.BlockSpec((TILE, c), lambda i: (i, 0))
        ftile = lambda c: pl.BlockSpec(
            (TILE, c), lambda i, _o=off // TILE: (i + _o, 0))

        evlo, evhi, epk = pl.pallas_call(
            _atoms_body,
            grid=(NT2,),
            in_specs=[ftile(H), ftile(3), ctile(16), ctile(HH), ctile(HH),
                      vec(128), vec(128), mat(128, 128), vec(128),
                      mat(H, H), mat(128, H), vec(H),
                      mat(H, H), mat(128, H), vec(H)],
            out_specs=[ctile(HH), ctile(HH),
                       pl.BlockSpec((TILE // 8, 128), lambda i: (i, 0))],
            out_shape=[jax.ShapeDtypeStruct((NA2, HH), jnp.float32),
                       jax.ShapeDtypeStruct((NA2, HH), jnp.float32),
                       jax.ShapeDtypeStruct((NA2 // 8, 128), jnp.float32)],
        )(af, atom_positions, centg, qlog, qhig,
          cen_pad, wid_pad, wg_pad, bg_pad,
          wka, wkb, p['bk'], wva, wvb, p['bv'])

        e16 = epk.reshape(NA2, 16)
        accs.append(_sc_ctx(evlo, evhi, e16, bid2, zlohi, zeros16, off))

    (alo0, ahi0, ae0), (alo1, ahi1, ae1) = accs

    updlo, updhi = pl.pallas_call(
        _upd_body,
        in_specs=[pl.BlockSpec((NC * NB, HH), lambda: (0, 0)),
                  pl.BlockSpec((NC * NB, HH), lambda: (0, 0)),
                  pl.BlockSpec((NC * NB, 16), lambda: (0, 0)),
                  pl.BlockSpec((NC * NB, HH), lambda: (0, 0)),
                  pl.BlockSpec((NC * NB, HH), lambda: (0, 0)),
                  pl.BlockSpec((NC * NB, 16), lambda: (0, 0)),
                  pl.BlockSpec((H, H), lambda: (0, 0)),
                  pl.BlockSpec((H,), lambda: (0,)),
                  pl.BlockSpec((H, H), lambda: (0, 0)),
                  pl.BlockSpec((H,), lambda: (0,))],
        out_specs=[pl.BlockSpec((NB, HH), lambda: (0, 0)),
                   pl.BlockSpec((NB, HH), lambda: (0, 0))],
        out_shape=[jax.ShapeDtypeStruct((NB, HH), jnp.float32),
                   jax.ShapeDtypeStruct((NB, HH), jnp.float32)],
    )(alo0, ahi0, ae0, alo1, ahi1, ae1,
      p['Wc1'], p['bc1'], p['Wc2'], p['bc2'])

    uglo, ughi = _sc_gather1(updlo, updhi, bid2, 0, N_ATOMS, PW, RPW)

    atile = lambda c: pl.BlockSpec((TILE, c), lambda i: (i, 0))
    out = pl.pallas_call(
        _final_body,
        grid=(NT,),
        in_specs=[atile(H), atile(HH), atile(HH),
                  mat(H, 2 * H), vec(2 * H), mat(2 * H, H), vec(H),
                  vec(H), vec(H), vec(H), vec(H)],
        out_specs=atile(H),
        out_shape=jax.ShapeDtypeStruct((N_ATOMS, H), jnp.float32),
    )(af, uglo, ughi,
      p['Wf1'], p['bf1'], p['Wf2'], p['bf2'],
      p['g1'], p['b1'], p['g2'], p['b2'])
    return out


# speedup vs baseline: 1.2864x; 1.2100x over previous
import functools
import math

import jax
import jax.numpy as jnp
from jax import lax
from jax.experimental import pallas as pl
from jax.experimental.pallas import tpu as pltpu
from jax.experimental.pallas import tpu_sc as plsc

N_ATOMS = 32768
NB = 1024
H = 256
HH = 128
H4 = 64
RBF = 16
EPS = 1e-5

NC = 2
NS = 16
NW = NC * NS
PW = N_ATOMS // NW
RPW = PW // 128
BPW = NB // NS

NCH = 2
NA2 = N_ATOMS // NCH
PW2 = NA2 // NW
RPW2 = PW2 // 128

TILE = 2048
NT = N_ATOMS // TILE
NT2 = NA2 // TILE

_MESH = functools.partial(
    plsc.VectorSubcoreMesh, core_axis_name="c", subcore_axis_name="s")


def _ln(x, g, b):
    mu = jnp.mean(x, axis=-1, keepdims=True)
    var = jnp.mean((x - mu) ** 2, axis=-1, keepdims=True)
    return (x - mu) * lax.rsqrt(var + EPS) * g + b


def _wid():
    return lax.axis_index("s") * NC + lax.axis_index("c")


def _sc_stats_body(posp_h, bid_h, zeros_h, out_h, idx_v, rows_v, zb_v, shared):
    c = lax.axis_index("c")
    s = lax.axis_index("s")
    wid = s * NC + c
    pltpu.sync_copy(zeros_h.at[pl.ds(s * BPW, BPW)], zb_v)
    pltpu.sync_copy(zb_v, shared.at[pl.ds(s * BPW, BPW)])
    plsc.subcore_barrier()
    pltpu.sync_copy(bid_h.at[pl.ds(wid * RPW, RPW)], idx_v)
    pltpu.sync_copy(posp_h.at[pl.ds(wid * PW, PW)], rows_v)
    for j in range(RPW):
        pltpu.sync_copy(rows_v.at[pl.ds(j * 128, 128)],
                        shared.at[idx_v.at[j]], add=True)
    plsc.subcore_barrier()
    pltpu.sync_copy(shared.at[pl.ds(s * BPW, BPW)], zb_v)
    pltpu.sync_copy(zb_v, out_h.at[pl.ds(c * NB + s * BPW, BPW)])


def _sc_stats(posp16, bid2, zeros16):
    return pl.kernel(
        _sc_stats_body,
        out_type=jax.ShapeDtypeStruct((NC * NB, 16), jnp.float32),
        mesh=_MESH(),
        compiler_params=pltpu.CompilerParams(use_tc_tiling_on_sc=False),
        scratch_types=[
            pltpu.VMEM((RPW, 128), jnp.int32),
            pltpu.VMEM((PW, 16), jnp.float32),
            pltpu.VMEM((BPW, 16), jnp.float32),
            pltpu.VMEM_SHARED((NB, 16), jnp.float32),
        ],
    )(posp16, bid2, zeros16)


def _pack_bf16(lo, hi):
    lo_u = lax.bitcast_convert_type(
        lax.convert_element_type(lo, jnp.bfloat16), jnp.uint16)
    hi_u = lax.bitcast_convert_type(
        lax.convert_element_type(hi, jnp.bfloat16), jnp.uint16)
    return lo_u.astype(jnp.uint32) | (hi_u.astype(jnp.uint32) << 16)


def _unpack_bf16(pk):
    lo = lax.bitcast_convert_type(pk << 16, jnp.float32)
    hi = lax.bitcast_convert_type(pk & jnp.uint32(0xFFFF0000), jnp.float32)
    return lo, hi


def _centq_body(stats_ref, bf_ref, wq_ref, bq_ref,
                cent_ref, qpk_ref):
    stats = stats_ref[:NB, :] + stats_ref[NB:, :]
    cnt = jnp.maximum(stats[:, 0:1], 1.0)
    lane = lax.broadcasted_iota(jnp.int32, (NB, 16), 1)
    keep = (lane >= 1) & (lane <= 3)
    cent_ref[...] = jnp.where(keep, stats / cnt, 0.0)
    q = jnp.dot(bf_ref[...], wq_ref[...],
                preferred_element_type=jnp.float32) + bq_ref[...]
    qpk_ref[...] = _pack_bf16(q[:, :HH], q[:, HH:])


def _make_gather2_body(off):
    orow = off // 128

    def body(cent_h, qpk_h, bid_h, centg_h, qpkg_h,
             idx_v, crow_v, b0_v, b1_v, csem, gsem, osem):
        wid = _wid()
        base = wid * PW2
        pltpu.sync_copy(bid_h.at[pl.ds(orow + wid * RPW2, RPW2)], idx_v)
        for j in range(RPW2):
            pltpu.async_copy(cent_h.at[idx_v.at[j]],
                             crow_v.at[pl.ds(j * 128, 128)], csem)
        bb = (b0_v, b1_v)
        pltpu.async_copy(qpk_h.at[idx_v.at[0]], b0_v, gsem)
        for j in range(RPW2):
            if j >= 1:
                pltpu.make_async_copy(
                    bb[(j - 1) % 2],
                    qpkg_h.at[pl.ds(base + (j - 1) * 128, 128)], osem).wait()
            if j + 1 < RPW2:
                pltpu.async_copy(qpk_h.at[idx_v.at[j + 1]],
                                 bb[(j + 1) % 2], gsem)
            pltpu.make_async_copy(qpk_h.at[idx_v.at[j]], bb[j % 2], gsem).wait()
            pltpu.async_copy(bb[j % 2],
                             qpkg_h.at[pl.ds(base + j * 128, 128)], osem)
        for j in range(RPW2):
            pltpu.make_async_copy(cent_h.at[idx_v.at[j]],
                                  crow_v.at[pl.ds(j * 128, 128)], csem).wait()
        pltpu.sync_copy(crow_v, centg_h.at[pl.ds(base, PW2)])
        pltpu.make_async_copy(
            bb[(RPW2 - 1) % 2],
            qpkg_h.at[pl.ds(base + (RPW2 - 1) * 128, 128)], osem).wait()

    return body


def _sc_gather2(cent16, qpk, bid2, off):
    return pl.kernel(
        _make_gather2_body(off),
        out_type=[jax.ShapeDtypeStruct((NA2, 16), jnp.float32),
                  jax.ShapeDtypeStruct((NA2, HH), jnp.uint32)],
        mesh=_MESH(),
        compiler_params=pltpu.CompilerParams(use_tc_tiling_on_sc=False),
        scratch_types=[
            pltpu.VMEM((RPW2, 128), jnp.int32),
            pltpu.VMEM((PW2, 16), jnp.float32),
            pltpu.VMEM((128, HH), jnp.uint32),
            pltpu.VMEM((128, HH), jnp.uint32),
            pltpu.SemaphoreType.DMA,
            pltpu.SemaphoreType.DMA,
            pltpu.SemaphoreType.DMA,
        ],
    )(cent16, qpk, bid2)


def _atoms_body(af_ref, pos_ref, centg_ref, qpkg_ref,
                cen_ref, wid_ref, wg_ref, bg_ref,
                wka_ref, wkb_ref, bk_ref, wva_ref, wvb_ref, bv_ref,
                evlo_ref, evhi_ref, epk_ref):
    pos = pos_ref[...]
    cg = centg_ref[...]
    dx = pos[:, 0:1] - cg[:, 1:2]
    dy = pos[:, 1:2] - cg[:, 2:3]
    dz = pos[:, 2:3] - cg[:, 3:4]
    dist = jnp.sqrt(dx * dx + dy * dy + dz * dz)
    d = dist - cen_ref[...]
    rbf = jnp.exp(-(d * d) / (2.0 * wid_ref[...] * wid_ref[...]))
    geom = jnp.dot(rbf, wg_ref[...],
                   preferred_element_type=jnp.float32) + bg_ref[...]

    af = af_ref[...]
    k = (jnp.dot(af, wka_ref[...], preferred_element_type=jnp.float32)
         + jnp.dot(geom, wkb_ref[...], preferred_element_type=jnp.float32)
         + bk_ref[...])
    v = (jnp.dot(af, wva_ref[...], preferred_element_type=jnp.float32)
         + jnp.dot(geom, wvb_ref[...], preferred_element_type=jnp.float32)
         + bv_ref[...])

    qlo, qhi = _unpack_bf16(qpkg_ref[...])
    s = (jnp.sum(qlo * k[:, :HH], axis=1)
         + jnp.sum(qhi * k[:, HH:], axis=1)) * (1.0 / math.sqrt(H))
    e = jnp.exp(s)
    ecol = e[:, None]
    evlo_ref[...] = ecol * v[:, :HH]
    evhi_ref[...] = ecol * v[:, HH:]
    e2 = e.reshape(TILE // 8, 8)
    grp = lax.broadcasted_iota(jnp.int32, (8, 128), 1) // 16
    row = lax.broadcasted_iota(jnp.int32, (8, 128), 0)
    sel = jnp.where(grp == row, 1.0, 0.0)
    epk_ref[...] = jnp.dot(e2, sel, preferred_element_type=jnp.float32)


def _make_ctx_body(off):
    orow = off // 128

    def body(evlo_h, evhi_h, e16_h, bid_h, zlohi_h, ze_h,
             acclo_h, acchi_h, acce_h,
             idx_v, l0_v, l1_v, h0_v, h1_v, erow_v, zb_v, zbe_v,
             shlo, shhi, she, lsem, ssem):
        c = lax.axis_index("c")
        s = lax.axis_index("s")
        wid = s * NC + c
        base = wid * PW2
        pltpu.sync_copy(zlohi_h.at[pl.ds(s * BPW, BPW)], zb_v)
        pltpu.sync_copy(zb_v, shlo.at[pl.ds(s * BPW, BPW)])
        pltpu.sync_copy(zb_v, shhi.at[pl.ds(s * BPW, BPW)])
        pltpu.sync_copy(ze_h.at[pl.ds(s * BPW, BPW)], zbe_v)
        pltpu.sync_copy(zbe_v, she.at[pl.ds(s * BPW, BPW)])
        pltpu.sync_copy(bid_h.at[pl.ds(orow + wid * RPW2, RPW2)], idx_v)
        pltpu.sync_copy(e16_h.at[pl.ds(base, PW2)], erow_v)
        plsc.subcore_barrier()
        lb = (l0_v, l1_v)
        hb = (h0_v, h1_v)
        pltpu.async_copy(evlo_h.at[pl.ds(base, 128)], l0_v, lsem)
        pltpu.async_copy(evhi_h.at[pl.ds(base, 128)], h0_v, lsem)
        for j in range(RPW2):
            pltpu.sync_copy(erow_v.at[pl.ds(j * 128, 128)],
                            she.at[idx_v.at[j]], add=True)
            if j >= 1:
                pltpu.make_async_copy(lb[(j - 1) % 2],
                                      shlo.at[idx_v.at[j - 1]], ssem).wait()
                pltpu.make_async_copy(hb[(j - 1) % 2],
                                      shhi.at[idx_v.at[j - 1]], ssem).wait()
            if j + 1 < RPW2:
                pltpu.async_copy(evlo_h.at[pl.ds(base + (j + 1) * 128, 128)],
                                 lb[(j + 1) % 2], lsem)
                pltpu.async_copy(evhi_h.at[pl.ds(base + (j + 1) * 128, 128)],
                                 hb[(j + 1) % 2], lsem)
            pltpu.make_async_copy(evlo_h.at[pl.ds(base + j * 128, 128)],
                                  lb[j % 2], lsem).wait()
            pltpu.make_async_copy(evhi_h.at[pl.ds(base + j * 128, 128)],
                                  hb[j % 2], lsem).wait()
            pltpu.async_copy(lb[j % 2], shlo.at[idx_v.at[j]], ssem, add=True)
            pltpu.async_copy(hb[j % 2], shhi.at[idx_v.at[j]], ssem, add=True)
        pltpu.make_async_copy(lb[(RPW2 - 1) % 2],
                              shlo.at[idx_v.at[RPW2 - 1]], ssem).wait()
        pltpu.make_async_copy(hb[(RPW2 - 1) % 2],
                              shhi.at[idx_v.at[RPW2 - 1]], ssem).wait()
        plsc.subcore_barrier()
        pltpu.sync_copy(shlo.at[pl.ds(s * BPW, BPW)], zb_v)
        pltpu.sync_copy(zb_v, acclo_h.at[pl.ds(c * NB + s * BPW, BPW)])
        pltpu.sync_copy(shhi.at[pl.ds(s * BPW, BPW)], zb_v)
        pltpu.sync_copy(zb_v, acchi_h.at[pl.ds(c * NB + s * BPW, BPW)])
        pltpu.sync_copy(she.at[pl.ds(s * BPW, BPW)], zbe_v)
        pltpu.sync_copy(zbe_v, acce_h.at[pl.ds(c * NB + s * BPW, BPW)])

    return body


def _sc_ctx(evlo, evhi, e16, bid2, zlohi, ze16, off):
    return pl.kernel(
        _make_ctx_body(off),
        out_type=[jax.ShapeDtypeStruct((NC * NB, HH), jnp.float32),
                  jax.ShapeDtypeStruct((NC * NB, HH), jnp.float32),
                  jax.ShapeDtypeStruct((NC * NB, 16), jnp.float32)],
        mesh=_MESH(),
        compiler_params=pltpu.CompilerParams(use_tc_tiling_on_sc=False),
        scratch_types=[
            pltpu.VMEM((RPW2, 128), jnp.int32),
            pltpu.VMEM((128, HH), jnp.float32),
            pltpu.VMEM((128, HH), jnp.float32),
            pltpu.VMEM((128, HH), jnp.float32),
            pltpu.VMEM((128, HH), jnp.float32),
            pltpu.VMEM((PW2, 16), jnp.float32),
            pltpu.VMEM((BPW, HH), jnp.float32),
            pltpu.VMEM((BPW, 16), jnp.float32),
            pltpu.VMEM_SHARED((NB, HH), jnp.float32),
            pltpu.VMEM_SHARED((NB, HH), jnp.float32),
            pltpu.VMEM_SHARED((NB, 16), jnp.float32),
            pltpu.SemaphoreType.DMA,
            pltpu.SemaphoreType.DMA,
        ],
    )(evlo, evhi, e16, bid2, zlohi, ze16)


def _upd_body(alo0_ref, ahi0_ref, ae0_ref, alo1_ref, ahi1_ref, ae1_ref,
              wc1_ref, bc1_ref, wc2_ref, bc2_ref, updpk_ref):
    ae = ae0_ref[:NB, 0:1] + ae0_ref[NB:, 0:1] \
        + ae1_ref[:NB, 0:1] + ae1_ref[NB:, 0:1]
    den = jnp.maximum(ae, 1e-30)
    ctx = jnp.concatenate(
        [alo0_ref[:NB, :] + alo0_ref[NB:, :]
         + alo1_ref[:NB, :] + alo1_ref[NB:, :],
         ahi0_ref[:NB, :] + ahi0_ref[NB:, :]
         + ahi1_ref[:NB, :] + ahi1_ref[NB:, :]], axis=1) / den
    h1 = jax.nn.relu(jnp.dot(ctx, wc1_ref[...],
                             preferred_element_type=jnp.float32) + bc1_ref[...])
    upd = jnp.dot(h1, wc2_ref[...],
                  preferred_element_type=jnp.float32) + bc2_ref[...]
    updpk_ref[...] = _pack_bf16(upd[:, :HH], upd[:, HH:])


def _sc_gather1_body(updpk_h, bid_h, ugpk_h,
                     idx_v, b0_v, b1_v, gsem, osem):
    wid = _wid()
    base = wid * PW
    pltpu.sync_copy(bid_h.at[pl.ds(wid * RPW, RPW)], idx_v)
    bb = (b0_v, b1_v)
    pltpu.async_copy(updpk_h.at[idx_v.at[0]], b0_v, gsem)
    for j in range(RPW):
        if j >= 1:
            pltpu.make_async_copy(
                bb[(j - 1) % 2],
                ugpk_h.at[pl.ds(base + (j - 1) * 128, 128)], osem).wait()
        if j + 1 < RPW:
            pltpu.async_copy(updpk_h.at[idx_v.at[j + 1]], bb[(j + 1) % 2], gsem)
        pltpu.make_async_copy(updpk_h.at[idx_v.at[j]], bb[j % 2], gsem).wait()
        pltpu.async_copy(bb[j % 2],
                         ugpk_h.at[pl.ds(base + j * 128, 128)], osem)
    pltpu.make_async_copy(
        bb[(RPW - 1) % 2],
        ugpk_h.at[pl.ds(base + (RPW - 1) * 128, 128)], osem).wait()


def _sc_gather1(updpk, bid2):
    return pl.kernel(
        _sc_gather1_body,
        out_type=jax.ShapeDtypeStruct((N_ATOMS, HH), jnp.uint32),
        mesh=_MESH(),
        compiler_params=pltpu.CompilerParams(use_tc_tiling_on_sc=False),
        scratch_types=[
            pltpu.VMEM((RPW, 128), jnp.int32),
            pltpu.VMEM((128, HH), jnp.uint32),
            pltpu.VMEM((128, HH), jnp.uint32),
            pltpu.SemaphoreType.DMA,
            pltpu.SemaphoreType.DMA,
        ],
    )(updpk, bid2)


def _final_body(af_ref, ugpk_ref,
                wf1_ref, bf1_ref, wf2_ref, bf2_ref,
                g1_ref, b1_ref, g2_ref, b2_ref, out_ref):
    uglo, ughi = _unpack_bf16(ugpk_ref[...])
    upd = jnp.concatenate([uglo, ughi], axis=1)
    x = _ln(af_ref[...] + upd, g1_ref[...], b1_ref[...])
    f = jax.nn.relu(jnp.dot(x, wf1_ref[...],
                            preferred_element_type=jnp.float32) + bf1_ref[...])
    f = jnp.dot(f, wf2_ref[...],
                preferred_element_type=jnp.float32) + bf2_ref[...]
    out_ref[...] = _ln(x + f, g2_ref[...], b2_ref[...])


def kernel(atom_features, atom_positions, block_features, params, block_id):
    p = params
    af = atom_features
    posp16 = jnp.concatenate(
        [jnp.ones((N_ATOMS, 1), jnp.float32), atom_positions,
         jnp.zeros((N_ATOMS, 12), jnp.float32)], axis=1)
    bid2 = block_id.reshape(N_ATOMS // 128, 128)
    zeros16 = jnp.zeros((NB, 16), jnp.float32)
    zlohi = jnp.zeros((NB, HH), jnp.float32)

    cen_pad = jnp.zeros((128,), jnp.float32).at[:RBF].set(p['centers'])
    wid_pad = jnp.ones((128,), jnp.float32).at[:RBF].set(p['widths'])
    wg_pad = jnp.zeros((128, 128), jnp.float32).at[:RBF, :H4].set(p['Wg'])
    bg_pad = jnp.zeros((128,), jnp.float32).at[:H4].set(p['bg'])
    wka, wkb = p['Wk'][:H], jnp.zeros((128, H), jnp.float32).at[:H4].set(p['Wk'][H:])
    wva, wvb = p['Wv'][:H], jnp.zeros((128, H), jnp.float32).at[:H4].set(p['Wv'][H:])

    stats = _sc_stats(posp16, bid2, zeros16)

    cent16, qpk = pl.pallas_call(
        _centq_body,
        in_specs=[pl.BlockSpec((NC * NB, 16), lambda: (0, 0)),
                  pl.BlockSpec((NB, H), lambda: (0, 0)),
                  pl.BlockSpec((H, H), lambda: (0, 0)),
                  pl.BlockSpec((H,), lambda: (0,))],
        out_specs=[pl.BlockSpec((NB, 16), lambda: (0, 0)),
                   pl.BlockSpec((NB, HH), lambda: (0, 0))],
        out_shape=[jax.ShapeDtypeStruct((NB, 16), jnp.float32),
                   jax.ShapeDtypeStruct((NB, HH), jnp.uint32)],
    )(stats, block_features, p['Wq'], p['bq'])

    vec = lambda n: pl.BlockSpec((n,), lambda i: (0,))
    mat = lambda a, b: pl.BlockSpec((a, b), lambda i: (0, 0))

    accs = []
    for ci in range(NCH):
        off = ci * NA2
        centg, qpkg = _sc_gather2(cent16, qpk, bid2, off)

        ctile = lambda c: pl.BlockSpec((TILE, c), lambda i: (i, 0))
        ftile = lambda c: pl.BlockSpec(
            (TILE, c), lambda i, _o=off // TILE: (i + _o, 0))

        evlo, evhi, epk = pl.pallas_call(
            _atoms_body,
            grid=(NT2,),
            in_specs=[ftile(H), ftile(3), ctile(16), ctile(HH),
                      vec(128), vec(128), mat(128, 128), vec(128),
                      mat(H, H), mat(128, H), vec(H),
                      mat(H, H), mat(128, H), vec(H)],
            out_specs=[ctile(HH), ctile(HH),
                       pl.BlockSpec((TILE // 8, 128), lambda i: (i, 0))],
            out_shape=[jax.ShapeDtypeStruct((NA2, HH), jnp.float32),
                       jax.ShapeDtypeStruct((NA2, HH), jnp.float32),
                       jax.ShapeDtypeStruct((NA2 // 8, 128), jnp.float32)],
        )(af, atom_positions, centg, qpkg,
          cen_pad, wid_pad, wg_pad, bg_pad,
          wka, wkb, p['bk'], wva, wvb, p['bv'])

        e16 = epk.reshape(NA2, 16)
        accs.append(_sc_ctx(evlo, evhi, e16, bid2, zlohi, zeros16, off))

    (alo0, ahi0, ae0), (alo1, ahi1, ae1) = accs

    updpk, = pl.pallas_call(
        _upd_body,
        in_specs=[pl.BlockSpec((NC * NB, HH), lambda: (0, 0)),
                  pl.BlockSpec((NC * NB, HH), lambda: (0, 0)),
                  pl.BlockSpec((NC * NB, 16), lambda: (0, 0)),
                  pl.BlockSpec((NC * NB, HH), lambda: (0, 0)),
                  pl.BlockSpec((NC * NB, HH), lambda: (0, 0)),
                  pl.BlockSpec((NC * NB, 16), lambda: (0, 0)),
                  pl.BlockSpec((H, H), lambda: (0, 0)),
                  pl.BlockSpec((H,), lambda: (0,)),
                  pl.BlockSpec((H, H), lambda: (0, 0)),
                  pl.BlockSpec((H,), lambda: (0,))],
        out_specs=[pl.BlockSpec((NB, HH), lambda: (0, 0))],
        out_shape=[jax.ShapeDtypeStruct((NB, HH), jnp.uint32)],
    )(alo0, ahi0, ae0, alo1, ahi1, ae1,
      p['Wc1'], p['bc1'], p['Wc2'], p['bc2'])

    ugpk = _sc_gather1(updpk, bid2)

    atile = lambda c: pl.BlockSpec((TILE, c), lambda i: (i, 0))
    out = pl.pallas_call(
        _final_body,
        grid=(NT,),
        in_specs=[atile(H), atile(HH),
                  mat(H, 2 * H), vec(2 * H), mat(2 * H, H), vec(H),
                  vec(H), vec(H), vec(H), vec(H)],
        out_specs=atile(H),
        out_shape=jax.ShapeDtypeStruct((N_ATOMS, H), jnp.float32),
    )(af, ugpk,
      p['Wf1'], p['bf1'], p['Wf2'], p['bf2'],
      p['g1'], p['b1'], p['g2'], p['b2'])
    return out
